# Initial kernel scaffold; baseline (speedup 1.0000x reference)
import jax
import jax.numpy as jnp
from jax.experimental import pallas as pl


def kernel(x, edge_index, params):
    raise NotImplementedError("write your pallas kernel here")


# speedup vs baseline: 2.2186x; 2.2186x over previous
import functools

import jax
import jax.numpy as jnp
from jax import lax
from jax.experimental import pallas as pl
from jax.experimental.pallas import tpu as pltpu
from jax.experimental.pallas import tpu_sc as plsc

N = 10000
E = 320000
D_IN = 128
HID = 256
NUM_LAYERS = 3

CHUNK = 128
N_TILES = 16
EP = 327680
N_CHUNKS = EP // CHUNK
CHUNKS_PER_TILE = N_CHUNKS // N_TILES
ACC_ROWS = 10112
ROWS_PER_TILE_INIT = ACC_ROWS // N_TILES
OUT_TILES = 10
ROWS_PER_TILE_OUT = N // OUT_TILES

_MM_PREC = lax.Precision.DEFAULT


def _make_seg_sum(split_edges):
    dh = 128
    mesh = plsc.VectorSubcoreMesh(core_axis_name="c", subcore_axis_name="s")
    chunks_per_tile = CHUNKS_PER_TILE // (2 if split_edges else 1)

    @functools.partial(
        pl.kernel,
        out_type=jax.ShapeDtypeStruct((2, N, dh), jnp.float32),
        mesh=mesh,
        scratch_types=[
            pltpu.VMEM((1, CHUNK), jnp.int32),
            pltpu.VMEM((1, CHUNK), jnp.int32),
            pltpu.VMEM((1, CHUNK), jnp.int32),
            pltpu.VMEM((CHUNK, dh), jnp.float32),
            pltpu.VMEM_SHARED((ACC_ROWS, dh), jnp.float32),
            pltpu.SemaphoreType.DMA,
        ],
    )
    def seg_sum(h_hbm, row_hbm, col_hbm, zero_hbm, out_hbm,
                rowbuf, rowbuf2, colbuf, gbuf, acc, sem):
        c = lax.axis_index("c")
        s = lax.axis_index("s")
        z0 = s * ROWS_PER_TILE_INIT
        pltpu.sync_copy(zero_hbm.at[pl.ds(z0, ROWS_PER_TILE_INIT)],
                        acc.at[pl.ds(z0, ROWS_PER_TILE_INIT)])
        plsc.subcore_barrier()

        if split_edges:
            base = (c * N_TILES + s) * (chunks_per_tile * CHUNK)
        else:
            base = s * (chunks_per_tile * CHUNK)

        @pl.loop(0, chunks_per_tile)
        def _(i):
            e0 = base + i * CHUNK
            pltpu.sync_copy(row_hbm.at[pl.ds(e0, CHUNK)], rowbuf.at[0])
            pltpu.sync_copy(col_hbm.at[pl.ds(e0, CHUNK)], colbuf.at[0])
            if split_edges:
                idx = rowbuf
            else:
                coff = c * N
                for q in range(CHUNK // 16):
                    rowbuf2[0, pl.ds(q * 16, 16)] = (
                        rowbuf[0, pl.ds(q * 16, 16)] + coff)
                idx = rowbuf2
            pltpu.async_copy(h_hbm.at[idx.at[0]], gbuf, sem).wait()
            pltpu.sync_copy(gbuf, acc.at[colbuf.at[0]], add=True)

        plsc.subcore_barrier()

        @pl.when(s < OUT_TILES)
        def _():
            o0 = s * ROWS_PER_TILE_OUT
            pltpu.sync_copy(acc.at[pl.ds(o0, ROWS_PER_TILE_OUT)],
                            out_hbm.at[c, pl.ds(o0, ROWS_PER_TILE_OUT)])

    return seg_sum


@functools.cache
def _seg_sum_kernel(split_edges):
    return _make_seg_sum(split_edges)


def _seg_sum_edges(*args):
    return _seg_sum_kernel(True)(*args)


def _seg_sum_feat(*args):
    return _seg_sum_kernel(False)(*args)


BR = 1000
NB = N // BR
_INV_N = 1.0 / N
_BN_EPS = 1e-5


def _matmul(a, b):
    return jnp.dot(a, b, preferred_element_type=jnp.float32,
                   precision=_MM_PREC)


def _mm_stats_body(partial_agg):
    def body(scale_ref, agg_ref, h_ref, w_ref, b_ref,
             t_ref, ssum_ref, ssq_ref):
        i = pl.program_id(0)
        s = scale_ref[0, 0]
        if partial_agg:
            z = agg_ref[0] + agg_ref[1] + s * h_ref[...]
        else:
            z = jnp.concatenate(
                [agg_ref[0] + s * h_ref[0], agg_ref[1] + s * h_ref[1]], axis=1)
        t = _matmul(z, w_ref[...]) + b_ref[...]
        t_ref[...] = t

        @pl.when(i == 0)
        def _():
            ssum_ref[...] = jnp.zeros_like(ssum_ref)
            ssq_ref[...] = jnp.zeros_like(ssq_ref)

        ssum_ref[...] += jnp.sum(t, axis=0, keepdims=True)
        ssq_ref[...] += jnp.sum(t * t, axis=0, keepdims=True)
    return body


def _bn_mm_stats_body(t_ref, ssum_ref, ssq_ref, g_ref, be_ref, w_ref, b_ref,
                      u_ref, usum_ref, usq_ref):
    i = pl.program_id(0)
    mu = ssum_ref[...] * _INV_N
    var = ssq_ref[...] * _INV_N - mu * mu
    tn = g_ref[...] * (t_ref[...] - mu) * lax.rsqrt(var + _BN_EPS) + be_ref[...]
    tn = jnp.maximum(tn, 0.0)
    u = _matmul(tn, w_ref[...]) + b_ref[...]
    u_ref[...] = u

    @pl.when(i == 0)
    def _():
        usum_ref[...] = jnp.zeros_like(usum_ref)
        usq_ref[...] = jnp.zeros_like(usq_ref)

    usum_ref[...] += jnp.sum(u, axis=0, keepdims=True)
    usq_ref[...] += jnp.sum(u * u, axis=0, keepdims=True)


def _bn_split_body(u_ref, usum_ref, usq_ref, g_ref, be_ref, out_ref):
    mu = usum_ref[...] * _INV_N
    var = usq_ref[...] * _INV_N - mu * mu
    un = g_ref[...] * (u_ref[...] - mu) * lax.rsqrt(var + _BN_EPS) + be_ref[...]
    un = jnp.maximum(un, 0.0)
    out_ref[0] = un[:, :HID // 2]
    out_ref[1] = un[:, HID // 2:]


def _bn_head_body(u_ref, usum_ref, usq_ref, g_ref, be_ref, hw_ref, hb_ref,
                  out_ref):
    mu = usum_ref[...] * _INV_N
    var = usq_ref[...] * _INV_N - mu * mu
    un = g_ref[...] * (u_ref[...] - mu) * lax.rsqrt(var + _BN_EPS) + be_ref[...]
    un = jnp.maximum(un, 0.0)
    out_ref[...] = _matmul(un, hw_ref[...]) + hb_ref[...]


def _vspec(block, imap):
    return pl.BlockSpec(block, imap, memory_space=pltpu.VMEM)


_ROWB = lambda i: (i, 0)
_CONST2 = lambda i: (0, 0)
_CONST3 = lambda i: (0, i, 0)
_STAT_SPEC = _vspec((1, HID), _CONST2)
_STAT_SHAPE = jax.ShapeDtypeStruct((1, HID), jnp.float32)


def _tc_layer(scale, agg, h, lp, partial_agg, head=None):
    d_in = D_IN if partial_agg else HID
    h_spec = (_vspec((BR, D_IN), _ROWB) if partial_agg
              else _vspec((2, BR, HID // 2), _CONST3))
    t, ssum, ssq = pl.pallas_call(
        _mm_stats_body(partial_agg),
        grid=(NB,),
        in_specs=[
            pl.BlockSpec(memory_space=pltpu.SMEM),
            _vspec((2, BR, HID // 2), _CONST3),
            h_spec,
            _vspec((d_in, HID), _CONST2),
            _STAT_SPEC,
        ],
        out_specs=[_vspec((BR, HID), _ROWB), _STAT_SPEC, _STAT_SPEC],
        out_shape=[jax.ShapeDtypeStruct((N, HID), jnp.float32),
                   _STAT_SHAPE, _STAT_SHAPE],
    )(scale, agg, h, lp['W1'], lp['b1'].reshape(1, HID))
    u, usum, usq = pl.pallas_call(
        _bn_mm_stats_body,
        grid=(NB,),
        in_specs=[
            _vspec((BR, HID), _ROWB), _STAT_SPEC, _STAT_SPEC,
            _STAT_SPEC, _STAT_SPEC,
            _vspec((HID, HID), _CONST2), _STAT_SPEC,
        ],
        out_specs=[_vspec((BR, HID), _ROWB), _STAT_SPEC, _STAT_SPEC],
        out_shape=[jax.ShapeDtypeStruct((N, HID), jnp.float32),
                   _STAT_SHAPE, _STAT_SHAPE],
    )(t, ssum, ssq, lp['bn1_g'].reshape(1, HID), lp['bn1_b'].reshape(1, HID),
      lp['W2'], lp['b2'].reshape(1, HID))
    if head is None:
        return pl.pallas_call(
            _bn_split_body,
            grid=(NB,),
            in_specs=[_vspec((BR, HID), _ROWB), _STAT_SPEC, _STAT_SPEC,
                      _STAT_SPEC, _STAT_SPEC],
            out_specs=_vspec((2, BR, HID // 2), _CONST3),
            out_shape=jax.ShapeDtypeStruct((2, N, HID // 2), jnp.float32),
        )(u, usum, usq, lp['bno_g'].reshape(1, HID),
          lp['bno_b'].reshape(1, HID))
    hw, hb = head
    return pl.pallas_call(
        _bn_head_body,
        grid=(NB,),
        in_specs=[_vspec((BR, HID), _ROWB), _STAT_SPEC, _STAT_SPEC,
                  _STAT_SPEC, _STAT_SPEC,
                  _vspec((HID, hw.shape[1]), _CONST2),
                  _vspec((1, hw.shape[1]), _CONST2)],
        out_specs=_vspec((BR, hw.shape[1]), _ROWB),
        out_shape=jax.ShapeDtypeStruct((N, hw.shape[1]), jnp.float32),
    )(u, usum, usq, lp['bno_g'].reshape(1, HID), lp['bno_b'].reshape(1, HID),
      hw, hb.reshape(1, hw.shape[1]))


def kernel(x, edge_index, params):
    row = edge_index[0].astype(jnp.int32)
    col = edge_index[1].astype(jnp.int32)
    pad = EP - E
    rowp = jnp.concatenate([row, jnp.zeros((pad,), jnp.int32)])
    colp = jnp.concatenate(
        [col, N + (jnp.arange(pad, dtype=jnp.int32) % (ACC_ROWS - N))])
    zeros128 = jnp.zeros((ACC_ROWS, 128), jnp.float32)

    out = None
    hcat = None
    for i in range(NUM_LAYERS):
        if i == 0:
            agg = _seg_sum_edges(x, rowp, colp, zeros128)
            h = x
        else:
            agg = _seg_sum_feat(hcat, rowp, colp, zeros128)
            h = hcat.reshape(2, N, HID // 2)
        scale = (1.0 + params['eps'][i]).reshape(1, 1)
        lp = params['layers'][i]
        if i < NUM_LAYERS - 1:
            hout = _tc_layer(scale, agg, h, lp, partial_agg=(i == 0))
            hcat = hout.reshape(2 * N, HID // 2)
        else:
            out = _tc_layer(scale, agg, h, lp, partial_agg=False,
                            head=(params['head_W'], params['head_b']))
    return out


# speedup vs baseline: 2.8364x; 1.2785x over previous
import functools

import jax
import jax.numpy as jnp
from jax import lax
from jax.experimental import pallas as pl
from jax.experimental.pallas import tpu as pltpu
from jax.experimental.pallas import tpu_sc as plsc

N = 10000
E = 320000
D_IN = 128
HID = 256
NUM_LAYERS = 3

CHUNK = 128
N_TILES = 16
EP = 327680
N_CHUNKS = EP // CHUNK
CHUNKS_PER_TILE = N_CHUNKS // N_TILES
ACC_ROWS = 10112
ROWS_PER_TILE_INIT = ACC_ROWS // N_TILES
OUT_TILES = 10
ROWS_PER_TILE_OUT = N // OUT_TILES

_MM_PREC = lax.Precision.DEFAULT


def _make_seg_sum(split_edges):
    dh = 128
    mesh = plsc.VectorSubcoreMesh(core_axis_name="c", subcore_axis_name="s")
    chunks_per_tile = CHUNKS_PER_TILE // (2 if split_edges else 1)

    @functools.partial(
        pl.kernel,
        out_type=jax.ShapeDtypeStruct((2, N, dh), jnp.float32),
        mesh=mesh,
        scratch_types=[
            pltpu.VMEM((1, CHUNK), jnp.int32),
            pltpu.VMEM((1, CHUNK), jnp.int32),
            pltpu.VMEM((1, CHUNK), jnp.int32),
            pltpu.VMEM((1, CHUNK), jnp.int32),
            pltpu.VMEM((1, CHUNK), jnp.int32),
            pltpu.VMEM((1, CHUNK), jnp.int32),
            pltpu.VMEM((CHUNK, dh), jnp.float32),
            pltpu.VMEM((CHUNK, dh), jnp.float32),
            pltpu.SemaphoreType.DMA,
            pltpu.SemaphoreType.DMA,
            pltpu.SemaphoreType.DMA,
            pltpu.SemaphoreType.DMA,
            pltpu.VMEM_SHARED((ACC_ROWS, dh), jnp.float32),
        ],
    )
    def seg_sum(h_hbm, row_hbm, col_hbm, zero_hbm, out_hbm,
                rb0, rb1, rr0, rr1, cb0, cb1, gb0, gb1,
                si0, si1, sg0, sg1, acc):
        rb = (rb0, rb1)
        rr = (rr0, rr1)
        cb = (cb0, cb1)
        gb = (gb0, gb1)
        si = (si0, si1)
        sg = (sg0, sg1)
        c = lax.axis_index("c")
        s = lax.axis_index("s")
        z0 = s * ROWS_PER_TILE_INIT
        pltpu.sync_copy(zero_hbm.at[pl.ds(z0, ROWS_PER_TILE_INIT)],
                        acc.at[pl.ds(z0, ROWS_PER_TILE_INIT)])
        plsc.subcore_barrier()

        if split_edges:
            base = (c * N_TILES + s) * (chunks_per_tile * CHUNK)
        else:
            base = s * (chunks_per_tile * CHUNK)
        coff = c * N

        def start_idx(e0, b):
            pltpu.async_copy(row_hbm.at[pl.ds(e0, CHUNK)], rb[b].at[0], si[b])
            pltpu.async_copy(col_hbm.at[pl.ds(e0, CHUNK)], cb[b].at[0], si[b])

        def wait_idx(b):
            pltpu.make_async_copy(
                row_hbm.at[pl.ds(0, CHUNK)], rb[b].at[0], si[b]).wait()
            pltpu.make_async_copy(
                col_hbm.at[pl.ds(0, CHUNK)], cb[b].at[0], si[b]).wait()

        def idxref(b):
            return rb[b] if split_edges else rr[b]

        def prep(b):
            if not split_edges:
                for q in range(CHUNK // 16):
                    rr[b][0, pl.ds(q * 16, 16)] = (
                        rb[b][0, pl.ds(q * 16, 16)] + coff)

        def gather_start(b):
            pltpu.async_copy(h_hbm.at[idxref(b).at[0]], gb[b], sg[b])

        def gather_wait(b):
            pltpu.make_async_copy(h_hbm.at[idxref(b).at[0]], gb[b],
                                  sg[b]).wait()

        def scatter(b):
            pltpu.sync_copy(gb[b], acc.at[cb[b].at[0]], add=True)

        def step(e_cur, b):
            bn = 1 - b
            wait_idx(bn)
            prep(bn)
            gather_wait(b)
            gather_start(bn)
            scatter(b)
            start_idx(e_cur + 2 * CHUNK, b)

        start_idx(base, 0)
        start_idx(base + CHUNK, 1)
        wait_idx(0)
        prep(0)
        gather_start(0)

        @pl.loop(0, (chunks_per_tile - 2) // 2)
        def _(k):
            e0 = base + (2 * k) * CHUNK
            step(e0, 0)
            step(e0 + CHUNK, 1)

        wait_idx(1)
        prep(1)
        gather_wait(0)
        gather_start(1)
        scatter(0)
        gather_wait(1)
        scatter(1)

        plsc.subcore_barrier()

        @pl.when(s < OUT_TILES)
        def _():
            o0 = s * ROWS_PER_TILE_OUT
            pltpu.sync_copy(acc.at[pl.ds(o0, ROWS_PER_TILE_OUT)],
                            out_hbm.at[c, pl.ds(o0, ROWS_PER_TILE_OUT)])

    return seg_sum


@functools.cache
def _seg_sum_kernel(split_edges):
    return _make_seg_sum(split_edges)


def _seg_sum_edges(*args):
    return _seg_sum_kernel(True)(*args)


def _seg_sum_feat(*args):
    return _seg_sum_kernel(False)(*args)


BR = 1000
NB = N // BR
_INV_N = 1.0 / N
_BN_EPS = 1e-5


def _matmul(a, b):
    return jnp.dot(a, b, preferred_element_type=jnp.float32,
                   precision=_MM_PREC)


def _mm_stats_body(partial_agg):
    def body(scale_ref, agg_ref, h_ref, w_ref, b_ref,
             t_ref, ssum_ref, ssq_ref):
        i = pl.program_id(0)
        s = scale_ref[0, 0]
        if partial_agg:
            z = agg_ref[0] + agg_ref[1] + s * h_ref[...]
        else:
            z = jnp.concatenate(
                [agg_ref[0] + s * h_ref[0], agg_ref[1] + s * h_ref[1]], axis=1)
        t = _matmul(z, w_ref[...]) + b_ref[...]
        t_ref[...] = t

        @pl.when(i == 0)
        def _():
            ssum_ref[...] = jnp.zeros_like(ssum_ref)
            ssq_ref[...] = jnp.zeros_like(ssq_ref)

        ssum_ref[...] += jnp.sum(t, axis=0, keepdims=True)
        ssq_ref[...] += jnp.sum(t * t, axis=0, keepdims=True)
    return body


def _bn_mm_stats_body(t_ref, ssum_ref, ssq_ref, g_ref, be_ref, w_ref, b_ref,
                      u_ref, usum_ref, usq_ref):
    i = pl.program_id(0)
    mu = ssum_ref[...] * _INV_N
    var = ssq_ref[...] * _INV_N - mu * mu
    tn = g_ref[...] * (t_ref[...] - mu) * lax.rsqrt(var + _BN_EPS) + be_ref[...]
    tn = jnp.maximum(tn, 0.0)
    u = _matmul(tn, w_ref[...]) + b_ref[...]
    u_ref[...] = u

    @pl.when(i == 0)
    def _():
        usum_ref[...] = jnp.zeros_like(usum_ref)
        usq_ref[...] = jnp.zeros_like(usq_ref)

    usum_ref[...] += jnp.sum(u, axis=0, keepdims=True)
    usq_ref[...] += jnp.sum(u * u, axis=0, keepdims=True)


def _bn_split_body(u_ref, usum_ref, usq_ref, g_ref, be_ref, out_ref):
    mu = usum_ref[...] * _INV_N
    var = usq_ref[...] * _INV_N - mu * mu
    un = g_ref[...] * (u_ref[...] - mu) * lax.rsqrt(var + _BN_EPS) + be_ref[...]
    un = jnp.maximum(un, 0.0)
    out_ref[0] = un[:, :HID // 2]
    out_ref[1] = un[:, HID // 2:]


def _bn_head_body(u_ref, usum_ref, usq_ref, g_ref, be_ref, hw_ref, hb_ref,
                  out_ref):
    mu = usum_ref[...] * _INV_N
    var = usq_ref[...] * _INV_N - mu * mu
    un = g_ref[...] * (u_ref[...] - mu) * lax.rsqrt(var + _BN_EPS) + be_ref[...]
    un = jnp.maximum(un, 0.0)
    out_ref[...] = _matmul(un, hw_ref[...]) + hb_ref[...]


def _vspec(block, imap):
    return pl.BlockSpec(block, imap, memory_space=pltpu.VMEM)


_ROWB = lambda i: (i, 0)
_CONST2 = lambda i: (0, 0)
_CONST3 = lambda i: (0, i, 0)
_STAT_SPEC = _vspec((1, HID), _CONST2)
_STAT_SHAPE = jax.ShapeDtypeStruct((1, HID), jnp.float32)


def _tc_layer(scale, agg, h, lp, partial_agg, head=None):
    d_in = D_IN if partial_agg else HID
    h_spec = (_vspec((BR, D_IN), _ROWB) if partial_agg
              else _vspec((2, BR, HID // 2), _CONST3))
    t, ssum, ssq = pl.pallas_call(
        _mm_stats_body(partial_agg),
        grid=(NB,),
        in_specs=[
            pl.BlockSpec(memory_space=pltpu.SMEM),
            _vspec((2, BR, HID // 2), _CONST3),
            h_spec,
            _vspec((d_in, HID), _CONST2),
            _STAT_SPEC,
        ],
        out_specs=[_vspec((BR, HID), _ROWB), _STAT_SPEC, _STAT_SPEC],
        out_shape=[jax.ShapeDtypeStruct((N, HID), jnp.float32),
                   _STAT_SHAPE, _STAT_SHAPE],
    )(scale, agg, h, lp['W1'], lp['b1'].reshape(1, HID))
    u, usum, usq = pl.pallas_call(
        _bn_mm_stats_body,
        grid=(NB,),
        in_specs=[
            _vspec((BR, HID), _ROWB), _STAT_SPEC, _STAT_SPEC,
            _STAT_SPEC, _STAT_SPEC,
            _vspec((HID, HID), _CONST2), _STAT_SPEC,
        ],
        out_specs=[_vspec((BR, HID), _ROWB), _STAT_SPEC, _STAT_SPEC],
        out_shape=[jax.ShapeDtypeStruct((N, HID), jnp.float32),
                   _STAT_SHAPE, _STAT_SHAPE],
    )(t, ssum, ssq, lp['bn1_g'].reshape(1, HID), lp['bn1_b'].reshape(1, HID),
      lp['W2'], lp['b2'].reshape(1, HID))
    if head is None:
        return pl.pallas_call(
            _bn_split_body,
            grid=(NB,),
            in_specs=[_vspec((BR, HID), _ROWB), _STAT_SPEC, _STAT_SPEC,
                      _STAT_SPEC, _STAT_SPEC],
            out_specs=_vspec((2, BR, HID // 2), _CONST3),
            out_shape=jax.ShapeDtypeStruct((2, N, HID // 2), jnp.float32),
        )(u, usum, usq, lp['bno_g'].reshape(1, HID),
          lp['bno_b'].reshape(1, HID))
    hw, hb = head
    return pl.pallas_call(
        _bn_head_body,
        grid=(NB,),
        in_specs=[_vspec((BR, HID), _ROWB), _STAT_SPEC, _STAT_SPEC,
                  _STAT_SPEC, _STAT_SPEC,
                  _vspec((HID, hw.shape[1]), _CONST2),
                  _vspec((1, hw.shape[1]), _CONST2)],
        out_specs=_vspec((BR, hw.shape[1]), _ROWB),
        out_shape=jax.ShapeDtypeStruct((N, hw.shape[1]), jnp.float32),
    )(u, usum, usq, lp['bno_g'].reshape(1, HID), lp['bno_b'].reshape(1, HID),
      hw, hb.reshape(1, hw.shape[1]))


def kernel(x, edge_index, params):
    row = edge_index[0].astype(jnp.int32)
    col = edge_index[1].astype(jnp.int32)
    pad = EP - E
    rowp = jnp.concatenate([row, jnp.zeros((pad,), jnp.int32)])
    colp = jnp.concatenate(
        [col, N + (jnp.arange(pad, dtype=jnp.int32) % (ACC_ROWS - N))])
    zeros128 = jnp.zeros((ACC_ROWS, 128), jnp.float32)

    out = None
    hcat = None
    for i in range(NUM_LAYERS):
        if i == 0:
            agg = _seg_sum_edges(x, rowp, colp, zeros128)
            h = x
        else:
            agg = _seg_sum_feat(hcat, rowp, colp, zeros128)
            h = hcat.reshape(2, N, HID // 2)
        scale = (1.0 + params['eps'][i]).reshape(1, 1)
        lp = params['layers'][i]
        if i < NUM_LAYERS - 1:
            hout = _tc_layer(scale, agg, h, lp, partial_agg=(i == 0))
            hcat = hout.reshape(2 * N, HID // 2)
        else:
            out = _tc_layer(scale, agg, h, lp, partial_agg=False,
                            head=(params['head_W'], params['head_b']))
    return out


# speedup vs baseline: 3.4819x; 1.2276x over previous
import functools

import jax
import jax.numpy as jnp
from jax import lax
from jax.experimental import pallas as pl
from jax.experimental.pallas import tpu as pltpu
from jax.experimental.pallas import tpu_sc as plsc

N = 10000
E = 320000
D_IN = 128
HID = 256
NUM_LAYERS = 3

CHUNK = 128
N_TILES = 16
EP = 327680
N_CHUNKS = EP // CHUNK
CHUNKS_PER_TILE = N_CHUNKS // N_TILES
ACC_ROWS = 12032
ROWS_PER_TILE_INIT = ACC_ROWS // N_TILES
OUT_TILES = 10
ROWS_PER_TILE_OUT = N // OUT_TILES

_MM_PREC = lax.Precision.DEFAULT


def _make_seg_sum(split_edges):
    dh = 128
    mesh = plsc.VectorSubcoreMesh(core_axis_name="c", subcore_axis_name="s")
    chunks_per_tile = CHUNKS_PER_TILE // (2 if split_edges else 1)

    @functools.partial(
        pl.kernel,
        out_type=jax.ShapeDtypeStruct((2, N, dh), jnp.float32),
        mesh=mesh,
        scratch_types=[
            pltpu.VMEM((1, CHUNK), jnp.int32),
            pltpu.VMEM((1, CHUNK), jnp.int32),
            pltpu.VMEM((1, CHUNK), jnp.int32),
            pltpu.VMEM((1, CHUNK), jnp.int32),
            pltpu.VMEM((1, CHUNK), jnp.int32),
            pltpu.VMEM((1, CHUNK), jnp.int32),
            pltpu.VMEM((CHUNK, dh), jnp.float32),
            pltpu.VMEM((CHUNK, dh), jnp.float32),
            pltpu.SemaphoreType.DMA,
            pltpu.SemaphoreType.DMA,
            pltpu.SemaphoreType.DMA,
            pltpu.SemaphoreType.DMA,
            pltpu.VMEM_SHARED((ACC_ROWS, dh), jnp.float32),
        ],
    )
    def seg_sum(h_hbm, row_hbm, col_hbm, zero_hbm, out_hbm,
                rb0, rb1, rr0, rr1, cb0, cb1, gb0, gb1,
                si0, si1, sg0, sg1, acc):
        rb = (rb0, rb1)
        rr = (rr0, rr1)
        cb = (cb0, cb1)
        gb = (gb0, gb1)
        si = (si0, si1)
        sg = (sg0, sg1)
        c = lax.axis_index("c")
        s = lax.axis_index("s")
        z0 = s * ROWS_PER_TILE_INIT
        pltpu.sync_copy(zero_hbm.at[pl.ds(z0, ROWS_PER_TILE_INIT)],
                        acc.at[pl.ds(z0, ROWS_PER_TILE_INIT)])
        plsc.subcore_barrier()

        if split_edges:
            base = (c * N_TILES + s) * (chunks_per_tile * CHUNK)
        else:
            base = s * (chunks_per_tile * CHUNK)
        coff = c * N

        def start_idx(e0, b):
            pltpu.async_copy(row_hbm.at[pl.ds(e0, CHUNK)], rb[b].at[0], si[b])
            pltpu.async_copy(col_hbm.at[pl.ds(e0, CHUNK)], cb[b].at[0], si[b])

        def wait_idx(b):
            pltpu.make_async_copy(
                row_hbm.at[pl.ds(0, CHUNK)], rb[b].at[0], si[b]).wait()
            pltpu.make_async_copy(
                col_hbm.at[pl.ds(0, CHUNK)], cb[b].at[0], si[b]).wait()

        def idxref(b):
            return rb[b] if split_edges else rr[b]

        def prep(b):
            if not split_edges:
                for q in range(CHUNK // 16):
                    rr[b][0, pl.ds(q * 16, 16)] = (
                        rb[b][0, pl.ds(q * 16, 16)] + coff)

        def gather_start(b):
            pltpu.async_copy(h_hbm.at[idxref(b).at[0]], gb[b], sg[b])

        def gather_wait(b):
            pltpu.make_async_copy(h_hbm.at[idxref(b).at[0]], gb[b],
                                  sg[b]).wait()

        def scatter(b):
            pltpu.sync_copy(gb[b], acc.at[cb[b].at[0]], add=True)

        def step(e_cur, b):
            bn = 1 - b
            wait_idx(bn)
            prep(bn)
            gather_wait(b)
            gather_start(bn)
            scatter(b)
            start_idx(e_cur + 2 * CHUNK, b)

        start_idx(base, 0)
        start_idx(base + CHUNK, 1)
        wait_idx(0)
        prep(0)
        gather_start(0)

        @pl.loop(0, (chunks_per_tile - 2) // 2)
        def _(k):
            e0 = base + (2 * k) * CHUNK
            step(e0, 0)
            step(e0 + CHUNK, 1)

        wait_idx(1)
        prep(1)
        gather_wait(0)
        gather_start(1)
        scatter(0)
        gather_wait(1)
        scatter(1)

        plsc.subcore_barrier()

        @pl.when(s < OUT_TILES)
        def _():
            o0 = s * ROWS_PER_TILE_OUT
            pltpu.sync_copy(acc.at[pl.ds(o0, ROWS_PER_TILE_OUT)],
                            out_hbm.at[c, pl.ds(o0, ROWS_PER_TILE_OUT)])

    return seg_sum


@functools.cache
def _seg_sum_kernel(split_edges):
    return _make_seg_sum(split_edges)


def _seg_sum_edges(*args):
    return _seg_sum_kernel(True)(*args)


def _seg_sum_feat(*args):
    return _seg_sum_kernel(False)(*args)


BR = 1000
NB = N // BR
_INV_N = 1.0 / N
_BN_EPS = 1e-5


def _matmul(a, b):
    return jnp.dot(a, b, preferred_element_type=jnp.float32,
                   precision=_MM_PREC)


def _mm_stats_body(partial_agg):
    def body(scale_ref, agg_ref, h_ref, w_ref, b_ref,
             t_ref, ssum_ref, ssq_ref):
        i = pl.program_id(0)
        s = scale_ref[0, 0]
        if partial_agg:
            z = agg_ref[0] + agg_ref[1] + s * h_ref[...]
        else:
            z = jnp.concatenate(
                [agg_ref[0] + s * h_ref[0], agg_ref[1] + s * h_ref[1]], axis=1)
        t = _matmul(z, w_ref[...]) + b_ref[...]
        t_ref[...] = t

        @pl.when(i == 0)
        def _():
            ssum_ref[...] = jnp.zeros_like(ssum_ref)
            ssq_ref[...] = jnp.zeros_like(ssq_ref)

        ssum_ref[...] += jnp.sum(t, axis=0, keepdims=True)
        ssq_ref[...] += jnp.sum(t * t, axis=0, keepdims=True)
    return body


def _bn_mm_stats_body(t_ref, ssum_ref, ssq_ref, g_ref, be_ref, w_ref, b_ref,
                      u_ref, usum_ref, usq_ref):
    i = pl.program_id(0)
    mu = ssum_ref[...] * _INV_N
    var = ssq_ref[...] * _INV_N - mu * mu
    tn = g_ref[...] * (t_ref[...] - mu) * lax.rsqrt(var + _BN_EPS) + be_ref[...]
    tn = jnp.maximum(tn, 0.0)
    u = _matmul(tn, w_ref[...]) + b_ref[...]
    u_ref[...] = u

    @pl.when(i == 0)
    def _():
        usum_ref[...] = jnp.zeros_like(usum_ref)
        usq_ref[...] = jnp.zeros_like(usq_ref)

    usum_ref[...] += jnp.sum(u, axis=0, keepdims=True)
    usq_ref[...] += jnp.sum(u * u, axis=0, keepdims=True)


def _bn_split_body(u_ref, usum_ref, usq_ref, g_ref, be_ref, out_ref):
    mu = usum_ref[...] * _INV_N
    var = usq_ref[...] * _INV_N - mu * mu
    un = g_ref[...] * (u_ref[...] - mu) * lax.rsqrt(var + _BN_EPS) + be_ref[...]
    un = jnp.maximum(un, 0.0)
    out_ref[0] = un[:, :HID // 2]
    out_ref[1] = un[:, HID // 2:]


def _bn_head_body(u_ref, usum_ref, usq_ref, g_ref, be_ref, hw_ref, hb_ref,
                  out_ref):
    mu = usum_ref[...] * _INV_N
    var = usq_ref[...] * _INV_N - mu * mu
    un = g_ref[...] * (u_ref[...] - mu) * lax.rsqrt(var + _BN_EPS) + be_ref[...]
    un = jnp.maximum(un, 0.0)
    out_ref[...] = _matmul(un, hw_ref[...]) + hb_ref[...]


def _vspec(block, imap):
    return pl.BlockSpec(block, imap, memory_space=pltpu.VMEM)


_ROWB = lambda i: (i, 0)
_CONST2 = lambda i: (0, 0)
_CONST3 = lambda i: (0, i, 0)
_STAT_SPEC = _vspec((1, HID), _CONST2)
_STAT_SHAPE = jax.ShapeDtypeStruct((1, HID), jnp.float32)


def _tc_layer(scale, agg, h, lp, partial_agg, head=None):
    d_in = D_IN if partial_agg else HID
    h_spec = (_vspec((BR, D_IN), _ROWB) if partial_agg
              else _vspec((2, BR, HID // 2), _CONST3))
    t, ssum, ssq = pl.pallas_call(
        _mm_stats_body(partial_agg),
        grid=(NB,),
        in_specs=[
            pl.BlockSpec(memory_space=pltpu.SMEM),
            _vspec((2, BR, HID // 2), _CONST3),
            h_spec,
            _vspec((d_in, HID), _CONST2),
            _STAT_SPEC,
        ],
        out_specs=[_vspec((BR, HID), _ROWB), _STAT_SPEC, _STAT_SPEC],
        out_shape=[jax.ShapeDtypeStruct((N, HID), jnp.float32),
                   _STAT_SHAPE, _STAT_SHAPE],
    )(scale, agg, h, lp['W1'], lp['b1'].reshape(1, HID))
    u, usum, usq = pl.pallas_call(
        _bn_mm_stats_body,
        grid=(NB,),
        in_specs=[
            _vspec((BR, HID), _ROWB), _STAT_SPEC, _STAT_SPEC,
            _STAT_SPEC, _STAT_SPEC,
            _vspec((HID, HID), _CONST2), _STAT_SPEC,
        ],
        out_specs=[_vspec((BR, HID), _ROWB), _STAT_SPEC, _STAT_SPEC],
        out_shape=[jax.ShapeDtypeStruct((N, HID), jnp.float32),
                   _STAT_SHAPE, _STAT_SHAPE],
    )(t, ssum, ssq, lp['bn1_g'].reshape(1, HID), lp['bn1_b'].reshape(1, HID),
      lp['W2'], lp['b2'].reshape(1, HID))
    if head is None:
        return pl.pallas_call(
            _bn_split_body,
            grid=(NB,),
            in_specs=[_vspec((BR, HID), _ROWB), _STAT_SPEC, _STAT_SPEC,
                      _STAT_SPEC, _STAT_SPEC],
            out_specs=_vspec((2, BR, HID // 2), _CONST3),
            out_shape=jax.ShapeDtypeStruct((2, N, HID // 2), jnp.float32),
        )(u, usum, usq, lp['bno_g'].reshape(1, HID),
          lp['bno_b'].reshape(1, HID))
    hw, hb = head
    return pl.pallas_call(
        _bn_head_body,
        grid=(NB,),
        in_specs=[_vspec((BR, HID), _ROWB), _STAT_SPEC, _STAT_SPEC,
                  _STAT_SPEC, _STAT_SPEC,
                  _vspec((HID, hw.shape[1]), _CONST2),
                  _vspec((1, hw.shape[1]), _CONST2)],
        out_specs=_vspec((BR, hw.shape[1]), _ROWB),
        out_shape=jax.ShapeDtypeStruct((N, hw.shape[1]), jnp.float32),
    )(u, usum, usq, lp['bno_g'].reshape(1, HID), lp['bno_b'].reshape(1, HID),
      hw, hb.reshape(1, hw.shape[1]))


def kernel(x, edge_index, params):
    row = edge_index[0].astype(jnp.int32)
    col = edge_index[1].astype(jnp.int32)

    def _pad_per_tile(n_seg):
        seg = E // n_seg
        pad = EP // n_seg - seg
        padrow = jnp.zeros((n_seg, pad), jnp.int32)
        padcol = N + (jnp.arange(n_seg * pad, dtype=jnp.int32)
                      % (ACC_ROWS - N)).reshape(n_seg, pad)
        r = jnp.concatenate([row.reshape(n_seg, seg), padrow], axis=1).reshape(-1)
        c = jnp.concatenate([col.reshape(n_seg, seg), padcol], axis=1).reshape(-1)
        return r, c

    rowp_e, colp_e = _pad_per_tile(2 * N_TILES)
    rowp_f, colp_f = _pad_per_tile(N_TILES)
    zeros128 = jnp.zeros((ACC_ROWS, 128), jnp.float32)

    out = None
    hcat = None
    for i in range(NUM_LAYERS):
        if i == 0:
            agg = _seg_sum_edges(x, rowp_e, colp_e, zeros128)
            h = x
        else:
            agg = _seg_sum_feat(hcat, rowp_f, colp_f, zeros128)
            h = hcat.reshape(2, N, HID // 2)
        scale = (1.0 + params['eps'][i]).reshape(1, 1)
        lp = params['layers'][i]
        if i < NUM_LAYERS - 1:
            hout = _tc_layer(scale, agg, h, lp, partial_agg=(i == 0))
            hcat = hout.reshape(2 * N, HID // 2)
        else:
            out = _tc_layer(scale, agg, h, lp, partial_agg=False,
                            head=(params['head_W'], params['head_b']))
    return out


# speedup vs baseline: 3.7869x; 1.0876x over previous
import functools

import jax
import jax.numpy as jnp
from jax import lax
from jax.experimental import pallas as pl
from jax.experimental.pallas import tpu as pltpu
from jax.experimental.pallas import tpu_sc as plsc

N = 10000
E = 320000
D_IN = 128
HID = 256
NUM_LAYERS = 3

CHUNK = 128
N_TILES = 16
EP = 327680
N_CHUNKS = EP // CHUNK
CHUNKS_PER_TILE = N_CHUNKS // N_TILES
ACC_ROWS = 12032
ROWS_PER_TILE_INIT = ACC_ROWS // N_TILES
OUT_TILES = 10
ROWS_PER_TILE_OUT = N // OUT_TILES

_MM_PREC = lax.Precision.DEFAULT


def _make_seg_sum(split_edges):
    dh = 128
    mesh = plsc.VectorSubcoreMesh(core_axis_name="c", subcore_axis_name="s")
    chunks_per_tile = CHUNKS_PER_TILE // (2 if split_edges else 1)

    @functools.partial(
        pl.kernel,
        out_type=jax.ShapeDtypeStruct((2, N, dh), jnp.float32),
        mesh=mesh,
        scratch_types=[
            pltpu.VMEM((1, CHUNK), jnp.int32),
            pltpu.VMEM((1, CHUNK), jnp.int32),
            pltpu.VMEM((1, CHUNK), jnp.int32),
            pltpu.VMEM((1, CHUNK), jnp.int32),
            pltpu.VMEM((1, CHUNK), jnp.int32),
            pltpu.VMEM((1, CHUNK), jnp.int32),
            pltpu.VMEM((CHUNK, dh), jnp.float32),
            pltpu.VMEM((CHUNK, dh), jnp.float32),
            pltpu.SemaphoreType.DMA,
            pltpu.SemaphoreType.DMA,
            pltpu.SemaphoreType.DMA,
            pltpu.SemaphoreType.DMA,
            pltpu.VMEM_SHARED((ACC_ROWS, dh), jnp.float32),
        ],
    )
    def seg_sum(h_hbm, row_hbm, col_hbm, zero_hbm, out_hbm,
                rb0, rb1, rr0, rr1, cb0, cb1, gb0, gb1,
                si0, si1, sg0, sg1, acc):
        rb = (rb0, rb1)
        rr = (rr0, rr1)
        cb = (cb0, cb1)
        gb = (gb0, gb1)
        si = (si0, si1)
        sg = (sg0, sg1)
        c = lax.axis_index("c")
        s = lax.axis_index("s")
        z0 = s * ROWS_PER_TILE_INIT
        pltpu.sync_copy(zero_hbm.at[pl.ds(z0, ROWS_PER_TILE_INIT)],
                        acc.at[pl.ds(z0, ROWS_PER_TILE_INIT)])
        plsc.subcore_barrier()

        if split_edges:
            base = (c * N_TILES + s) * (chunks_per_tile * CHUNK)
        else:
            base = s * (chunks_per_tile * CHUNK)
        coff = c * N

        def start_idx(e0, b):
            pltpu.async_copy(row_hbm.at[pl.ds(e0, CHUNK)], rb[b].at[0], si[b])
            pltpu.async_copy(col_hbm.at[pl.ds(e0, CHUNK)], cb[b].at[0], si[b])

        def wait_idx(b):
            pltpu.make_async_copy(
                row_hbm.at[pl.ds(0, CHUNK)], rb[b].at[0], si[b]).wait()
            pltpu.make_async_copy(
                col_hbm.at[pl.ds(0, CHUNK)], cb[b].at[0], si[b]).wait()

        def idxref(b):
            return rr[b]

        def prep(b):
            for q in range(CHUNK // 16):
                rr[b][0, pl.ds(q * 16, 16)] = (
                    rb[b][0, pl.ds(q * 16, 16)] + coff)

        def gather_start(b):
            pltpu.async_copy(h_hbm.at[idxref(b).at[0]], gb[b], sg[b])

        def gather_wait(b):
            pltpu.make_async_copy(h_hbm.at[idxref(b).at[0]], gb[b],
                                  sg[b]).wait()

        def scatter(b):
            pltpu.sync_copy(gb[b], acc.at[cb[b].at[0]], add=True)

        def step(e_cur, b):
            bn = 1 - b
            wait_idx(bn)
            prep(bn)
            gather_wait(b)
            gather_start(bn)
            scatter(b)
            start_idx(e_cur + 2 * CHUNK, b)

        start_idx(base, 0)
        start_idx(base + CHUNK, 1)
        wait_idx(0)
        prep(0)
        gather_start(0)

        @pl.loop(0, (chunks_per_tile - 2) // 2)
        def _(k):
            e0 = base + (2 * k) * CHUNK
            step(e0, 0)
            step(e0 + CHUNK, 1)

        wait_idx(1)
        prep(1)
        gather_wait(0)
        gather_start(1)
        scatter(0)
        gather_wait(1)
        scatter(1)

        plsc.subcore_barrier()

        @pl.when(s < OUT_TILES)
        def _():
            o0 = s * ROWS_PER_TILE_OUT
            pltpu.sync_copy(acc.at[pl.ds(o0, ROWS_PER_TILE_OUT)],
                            out_hbm.at[c, pl.ds(o0, ROWS_PER_TILE_OUT)])

    return seg_sum


@functools.cache
def _seg_sum_kernel(split_edges):
    return _make_seg_sum(split_edges)


def _seg_sum_edges(*args):
    return _seg_sum_kernel(True)(*args)


def _seg_sum_feat(*args):
    return _seg_sum_kernel(False)(*args)


BR = 1000
NB = N // BR
_INV_N = 1.0 / N
_BN_EPS = 1e-5


def _matmul(a, b):
    return jnp.dot(a, b, preferred_element_type=jnp.float32,
                   precision=_MM_PREC)


def _mm_stats_body(partial_agg):
    def body(scale_ref, agg_ref, h_ref, w_ref, b_ref,
             t_ref, ssum_ref, ssq_ref):
        i = pl.program_id(0)
        s = scale_ref[0, 0]
        if partial_agg:
            z = agg_ref[0] + agg_ref[1] + s * h_ref[...]
        else:
            z = jnp.concatenate(
                [agg_ref[0] + s * h_ref[0], agg_ref[1] + s * h_ref[1]], axis=1)
        t = _matmul(z, w_ref[...]) + b_ref[...]
        t_ref[...] = t

        @pl.when(i == 0)
        def _():
            ssum_ref[...] = jnp.zeros_like(ssum_ref)
            ssq_ref[...] = jnp.zeros_like(ssq_ref)

        ssum_ref[...] += jnp.sum(t, axis=0, keepdims=True)
        ssq_ref[...] += jnp.sum(t * t, axis=0, keepdims=True)
    return body


def _bn_mm_stats_body(t_ref, ssum_ref, ssq_ref, g_ref, be_ref, w_ref, b_ref,
                      u_ref, usum_ref, usq_ref):
    i = pl.program_id(0)
    mu = ssum_ref[...] * _INV_N
    var = ssq_ref[...] * _INV_N - mu * mu
    tn = g_ref[...] * (t_ref[...] - mu) * lax.rsqrt(var + _BN_EPS) + be_ref[...]
    tn = jnp.maximum(tn, 0.0)
    u = _matmul(tn, w_ref[...]) + b_ref[...]
    u_ref[...] = u

    @pl.when(i == 0)
    def _():
        usum_ref[...] = jnp.zeros_like(usum_ref)
        usq_ref[...] = jnp.zeros_like(usq_ref)

    usum_ref[...] += jnp.sum(u, axis=0, keepdims=True)
    usq_ref[...] += jnp.sum(u * u, axis=0, keepdims=True)


def _bn_split_body(u_ref, usum_ref, usq_ref, g_ref, be_ref, out_ref):
    mu = usum_ref[...] * _INV_N
    var = usq_ref[...] * _INV_N - mu * mu
    un = g_ref[...] * (u_ref[...] - mu) * lax.rsqrt(var + _BN_EPS) + be_ref[...]
    un = jnp.maximum(un, 0.0)
    out_ref[0] = un[:, :HID // 2]
    out_ref[1] = un[:, HID // 2:]


def _bn_head_body(u_ref, usum_ref, usq_ref, g_ref, be_ref, hw_ref, hb_ref,
                  out_ref):
    mu = usum_ref[...] * _INV_N
    var = usq_ref[...] * _INV_N - mu * mu
    un = g_ref[...] * (u_ref[...] - mu) * lax.rsqrt(var + _BN_EPS) + be_ref[...]
    un = jnp.maximum(un, 0.0)
    out_ref[...] = _matmul(un, hw_ref[...]) + hb_ref[...]


def _vspec(block, imap):
    return pl.BlockSpec(block, imap, memory_space=pltpu.VMEM)


_ROWB = lambda i: (i, 0)
_CONST2 = lambda i: (0, 0)
_CONST3 = lambda i: (0, i, 0)
_STAT_SPEC = _vspec((1, HID), _CONST2)
_STAT_SHAPE = jax.ShapeDtypeStruct((1, HID), jnp.float32)


def _tc_layer(scale, agg, h, lp, partial_agg, head=None):
    d_in = D_IN if partial_agg else HID
    h_spec = (_vspec((BR, D_IN), _ROWB) if partial_agg
              else _vspec((2, BR, HID // 2), _CONST3))
    t, ssum, ssq = pl.pallas_call(
        _mm_stats_body(partial_agg),
        grid=(NB,),
        in_specs=[
            pl.BlockSpec(memory_space=pltpu.SMEM),
            _vspec((2, BR, HID // 2), _CONST3),
            h_spec,
            _vspec((d_in, HID), _CONST2),
            _STAT_SPEC,
        ],
        out_specs=[_vspec((BR, HID), _ROWB), _STAT_SPEC, _STAT_SPEC],
        out_shape=[jax.ShapeDtypeStruct((N, HID), jnp.float32),
                   _STAT_SHAPE, _STAT_SHAPE],
    )(scale, agg, h, lp['W1'], lp['b1'].reshape(1, HID))
    u, usum, usq = pl.pallas_call(
        _bn_mm_stats_body,
        grid=(NB,),
        in_specs=[
            _vspec((BR, HID), _ROWB), _STAT_SPEC, _STAT_SPEC,
            _STAT_SPEC, _STAT_SPEC,
            _vspec((HID, HID), _CONST2), _STAT_SPEC,
        ],
        out_specs=[_vspec((BR, HID), _ROWB), _STAT_SPEC, _STAT_SPEC],
        out_shape=[jax.ShapeDtypeStruct((N, HID), jnp.float32),
                   _STAT_SHAPE, _STAT_SHAPE],
    )(t, ssum, ssq, lp['bn1_g'].reshape(1, HID), lp['bn1_b'].reshape(1, HID),
      lp['W2'], lp['b2'].reshape(1, HID))
    if head is None:
        return pl.pallas_call(
            _bn_split_body,
            grid=(NB,),
            in_specs=[_vspec((BR, HID), _ROWB), _STAT_SPEC, _STAT_SPEC,
                      _STAT_SPEC, _STAT_SPEC],
            out_specs=_vspec((2, BR, HID // 2), _CONST3),
            out_shape=jax.ShapeDtypeStruct((2, N, HID // 2), jnp.float32),
        )(u, usum, usq, lp['bno_g'].reshape(1, HID),
          lp['bno_b'].reshape(1, HID))
    hw, hb = head
    return pl.pallas_call(
        _bn_head_body,
        grid=(NB,),
        in_specs=[_vspec((BR, HID), _ROWB), _STAT_SPEC, _STAT_SPEC,
                  _STAT_SPEC, _STAT_SPEC,
                  _vspec((HID, hw.shape[1]), _CONST2),
                  _vspec((1, hw.shape[1]), _CONST2)],
        out_specs=_vspec((BR, hw.shape[1]), _ROWB),
        out_shape=jax.ShapeDtypeStruct((N, hw.shape[1]), jnp.float32),
    )(u, usum, usq, lp['bno_g'].reshape(1, HID), lp['bno_b'].reshape(1, HID),
      hw, hb.reshape(1, hw.shape[1]))


def kernel(x, edge_index, params):
    row = edge_index[0].astype(jnp.int32)
    col = edge_index[1].astype(jnp.int32)

    def _pad_per_tile(n_seg):
        seg = E // n_seg
        pad = EP // n_seg - seg
        padrow = jnp.zeros((n_seg, pad), jnp.int32)
        padcol = N + (jnp.arange(n_seg * pad, dtype=jnp.int32)
                      % (ACC_ROWS - N)).reshape(n_seg, pad)
        r = jnp.concatenate([row.reshape(n_seg, seg), padrow], axis=1).reshape(-1)
        c = jnp.concatenate([col.reshape(n_seg, seg), padcol], axis=1).reshape(-1)
        return r, c

    rowp_e, colp_e = _pad_per_tile(2 * N_TILES)
    rowp_f, colp_f = _pad_per_tile(N_TILES)
    zeros128 = jnp.zeros((ACC_ROWS, 128), jnp.float32)

    out = None
    hcat = None
    for i in range(NUM_LAYERS):
        if i == 0:
            x2 = jnp.concatenate([x, x], axis=0)
            agg = _seg_sum_edges(x2, rowp_e, colp_e, zeros128)
            h = x
        else:
            agg = _seg_sum_feat(hcat, rowp_f, colp_f, zeros128)
            h = hcat.reshape(2, N, HID // 2)
        scale = (1.0 + params['eps'][i]).reshape(1, 1)
        lp = params['layers'][i]
        if i < NUM_LAYERS - 1:
            hout = _tc_layer(scale, agg, h, lp, partial_agg=(i == 0))
            hcat = hout.reshape(2 * N, HID // 2)
        else:
            out = _tc_layer(scale, agg, h, lp, partial_agg=False,
                            head=(params['head_W'], params['head_b']))
    return out


# speedup vs baseline: 3.8764x; 1.0236x over previous
import functools

import jax
import jax.numpy as jnp
from jax import lax
from jax.experimental import pallas as pl
from jax.experimental.pallas import tpu as pltpu
from jax.experimental.pallas import tpu_sc as plsc

N = 10000
E = 320000
D_IN = 128
HID = 256
NUM_LAYERS = 3

CHUNK = 128
N_TILES = 16
EP = 327680
N_CHUNKS = EP // CHUNK
CHUNKS_PER_TILE = N_CHUNKS // N_TILES
ACC_ROWS = 12032
ROWS_PER_TILE_INIT = ACC_ROWS // N_TILES
OUT_TILES = 10
ROWS_PER_TILE_OUT = N // OUT_TILES

_MM_PREC = lax.Precision.DEFAULT


def _make_seg_sum(split_edges):
    dh = 128
    mesh = plsc.VectorSubcoreMesh(core_axis_name="c", subcore_axis_name="s")
    chunks_per_tile = CHUNKS_PER_TILE // (2 if split_edges else 1)

    @functools.partial(
        pl.kernel,
        out_type=jax.ShapeDtypeStruct((2, N, dh), jnp.float32),
        mesh=mesh,
        scratch_types=[
            pltpu.VMEM((1, CHUNK), jnp.int32),
            pltpu.VMEM((1, CHUNK), jnp.int32),
            pltpu.VMEM((1, CHUNK), jnp.int32),
            pltpu.VMEM((1, CHUNK), jnp.int32),
            pltpu.VMEM((1, CHUNK), jnp.int32),
            pltpu.VMEM((1, CHUNK), jnp.int32),
            pltpu.VMEM((CHUNK, dh), jnp.float32),
            pltpu.VMEM((CHUNK, dh), jnp.float32),
            pltpu.SemaphoreType.DMA,
            pltpu.SemaphoreType.DMA,
            pltpu.SemaphoreType.DMA,
            pltpu.SemaphoreType.DMA,
            pltpu.VMEM_SHARED((ACC_ROWS, dh), jnp.float32),
        ],
    )
    def seg_sum(h_hbm, row_hbm, col_hbm, zero_hbm, out_hbm,
                rb0, rb1, rr0, rr1, cb0, cb1, gb0, gb1,
                si0, si1, sg0, sg1, acc):
        rb = (rb0, rb1)
        rr = (rr0, rr1)
        cb = (cb0, cb1)
        gb = (gb0, gb1)
        si = (si0, si1)
        sg = (sg0, sg1)
        c = lax.axis_index("c")
        s = lax.axis_index("s")
        z0 = s * ROWS_PER_TILE_INIT
        pltpu.sync_copy(zero_hbm.at[pl.ds(z0, ROWS_PER_TILE_INIT)],
                        acc.at[pl.ds(z0, ROWS_PER_TILE_INIT)])
        plsc.subcore_barrier()

        if split_edges:
            base = (c * N_TILES + s) * (chunks_per_tile * CHUNK)
        else:
            base = s * (chunks_per_tile * CHUNK)
        coff = c * N

        def start_idx(e0, b):
            pltpu.async_copy(row_hbm.at[pl.ds(e0, CHUNK)], rb[b].at[0], si[b])
            pltpu.async_copy(col_hbm.at[pl.ds(e0, CHUNK)], cb[b].at[0], si[b])

        def wait_idx(b):
            pltpu.make_async_copy(
                row_hbm.at[pl.ds(0, CHUNK)], rb[b].at[0], si[b]).wait()
            pltpu.make_async_copy(
                col_hbm.at[pl.ds(0, CHUNK)], cb[b].at[0], si[b]).wait()

        def idxref(b):
            return rr[b]

        def prep(b):
            for q in range(CHUNK // 16):
                rr[b][0, pl.ds(q * 16, 16)] = (
                    rb[b][0, pl.ds(q * 16, 16)] + coff)

        def gather_start(b):
            pltpu.async_copy(h_hbm.at[idxref(b).at[0]], gb[b], sg[b])

        def gather_wait(b):
            pltpu.make_async_copy(h_hbm.at[idxref(b).at[0]], gb[b],
                                  sg[b]).wait()

        def scatter(b):
            pltpu.sync_copy(gb[b], acc.at[cb[b].at[0]], add=True)

        def step(e_cur, b):
            bn = 1 - b
            wait_idx(bn)
            prep(bn)
            gather_wait(b)
            gather_start(bn)
            scatter(b)
            start_idx(e_cur + 2 * CHUNK, b)

        start_idx(base, 0)
        start_idx(base + CHUNK, 1)
        wait_idx(0)
        prep(0)
        gather_start(0)

        @pl.loop(0, (chunks_per_tile - 2) // 2)
        def _(k):
            e0 = base + (2 * k) * CHUNK
            step(e0, 0)
            step(e0 + CHUNK, 1)

        wait_idx(1)
        prep(1)
        gather_wait(0)
        gather_start(1)
        scatter(0)
        gather_wait(1)
        scatter(1)

        plsc.subcore_barrier()

        @pl.when(s < OUT_TILES)
        def _():
            o0 = s * ROWS_PER_TILE_OUT
            pltpu.sync_copy(acc.at[pl.ds(o0, ROWS_PER_TILE_OUT)],
                            out_hbm.at[c, pl.ds(o0, ROWS_PER_TILE_OUT)])

    return seg_sum


@functools.cache
def _seg_sum_kernel(split_edges):
    return _make_seg_sum(split_edges)


def _seg_sum_edges(*args):
    return _seg_sum_kernel(True)(*args)


def _seg_sum_feat(*args):
    return _seg_sum_kernel(False)(*args)


BR = 1000
NB = N // BR
_INV_N = 1.0 / N
_BN_EPS = 1e-5


def _matmul(a, b):
    return jnp.dot(a, b, preferred_element_type=jnp.float32,
                   precision=_MM_PREC)


def _vspec(block, imap):
    return pl.BlockSpec(block, imap, memory_space=pltpu.VMEM)


def _fused_layer_body(partial_agg, with_head):
    def body(*refs):
        if with_head:
            (scale_ref, agg_ref, h_ref, w1_ref, b1_ref, g1_ref, be1_ref,
             w2_ref, b2_ref, go_ref, bo_ref, hw_ref, hb_ref, out_ref,
             t_scr, u_scr, s0, s1, s2, s3) = refs
        else:
            (scale_ref, agg_ref, h_ref, w1_ref, b1_ref, g1_ref, be1_ref,
             w2_ref, b2_ref, go_ref, bo_ref, out_ref,
             t_scr, u_scr, s0, s1, s2, s3) = refs
        p = pl.program_id(0)
        i = pl.program_id(1)
        rows = pl.ds(i * BR, BR)

        @pl.when(p == 0)
        def _():
            s = scale_ref[0, 0]
            if partial_agg:
                z = agg_ref[0] + agg_ref[1] + s * h_ref[...]
            else:
                z = jnp.concatenate(
                    [agg_ref[0] + s * h_ref[0], agg_ref[1] + s * h_ref[1]],
                    axis=1)
            t = _matmul(z, w1_ref[...]) + b1_ref[...]
            t_scr[rows] = t

            @pl.when(i == 0)
            def _():
                s0[...] = jnp.zeros_like(s0)
                s1[...] = jnp.zeros_like(s1)

            s0[...] += jnp.sum(t, axis=0, keepdims=True)
            s1[...] += jnp.sum(t * t, axis=0, keepdims=True)

        @pl.when(p == 1)
        def _():
            mu = s0[...] * _INV_N
            var = s1[...] * _INV_N - mu * mu
            tn = (g1_ref[...] * (t_scr[rows] - mu) * lax.rsqrt(var + _BN_EPS)
                  + be1_ref[...])
            tn = jnp.maximum(tn, 0.0)
            u = _matmul(tn, w2_ref[...]) + b2_ref[...]
            u_scr[rows] = u

            @pl.when(i == 0)
            def _():
                s2[...] = jnp.zeros_like(s2)
                s3[...] = jnp.zeros_like(s3)

            s2[...] += jnp.sum(u, axis=0, keepdims=True)
            s3[...] += jnp.sum(u * u, axis=0, keepdims=True)

        @pl.when(p == 2)
        def _():
            mu = s2[...] * _INV_N
            var = s3[...] * _INV_N - mu * mu
            un = (go_ref[...] * (u_scr[rows] - mu) * lax.rsqrt(var + _BN_EPS)
                  + bo_ref[...])
            un = jnp.maximum(un, 0.0)
            if with_head:
                out_ref[...] = _matmul(un, hw_ref[...]) + hb_ref[...]
            else:
                out_ref[0] = un[:, :HID // 2]
                out_ref[1] = un[:, HID // 2:]
    return body


def _rowb_p0(p, i):
    return (0, jnp.where(p == 0, i, 0), 0)


def _rowb_p0_2d(p, i):
    return (jnp.where(p == 0, i, 0), 0)


def _rowb_p2(p, i):
    return (0, jnp.where(p == 2, i, 0), 0)


def _rowb_p2_2d(p, i):
    return (jnp.where(p == 2, i, 0), 0)


_CONST2 = lambda p, i: (0, 0)
_STAT = pltpu.VMEM((1, HID), jnp.float32)


def _tc_layer(scale, agg, h, lp, partial_agg, head=None):
    d_in = D_IN if partial_agg else HID
    h_spec = (_vspec((BR, D_IN), _rowb_p0_2d) if partial_agg
              else _vspec((2, BR, HID // 2), _rowb_p0))
    args = [scale, agg, h,
            lp['W1'], lp['b1'].reshape(1, HID),
            lp['bn1_g'].reshape(1, HID), lp['bn1_b'].reshape(1, HID),
            lp['W2'], lp['b2'].reshape(1, HID),
            lp['bno_g'].reshape(1, HID), lp['bno_b'].reshape(1, HID)]
    in_specs = [
        pl.BlockSpec(memory_space=pltpu.SMEM),
        _vspec((2, BR, HID // 2), _rowb_p0),
        h_spec,
        _vspec((d_in, HID), _CONST2), _vspec((1, HID), _CONST2),
        _vspec((1, HID), _CONST2), _vspec((1, HID), _CONST2),
        _vspec((HID, HID), _CONST2), _vspec((1, HID), _CONST2),
        _vspec((1, HID), _CONST2), _vspec((1, HID), _CONST2),
    ]
    if head is None:
        out_spec = _vspec((2, BR, HID // 2), _rowb_p2)
        out_shape = jax.ShapeDtypeStruct((2, N, HID // 2), jnp.float32)
    else:
        hw, hb = head
        args += [hw, hb.reshape(1, hw.shape[1])]
        in_specs += [_vspec((HID, hw.shape[1]), _CONST2),
                     _vspec((1, hw.shape[1]), _CONST2)]
        out_spec = _vspec((BR, hw.shape[1]), _rowb_p2_2d)
        out_shape = jax.ShapeDtypeStruct((N, hw.shape[1]), jnp.float32)
    return pl.pallas_call(
        _fused_layer_body(partial_agg, head is not None),
        grid=(3, NB),
        in_specs=in_specs,
        out_specs=out_spec,
        out_shape=out_shape,
        scratch_shapes=[
            pltpu.VMEM((N, HID), jnp.float32),
            pltpu.VMEM((N, HID), jnp.float32),
            _STAT, _STAT, _STAT, _STAT,
        ],
    )(*args)


def kernel(x, edge_index, params):
    row = edge_index[0].astype(jnp.int32)
    col = edge_index[1].astype(jnp.int32)

    def _pad_per_tile(n_seg):
        seg = E // n_seg
        pad = EP // n_seg - seg
        padrow = jnp.zeros((n_seg, pad), jnp.int32)
        padcol = N + (jnp.arange(n_seg * pad, dtype=jnp.int32)
                      % (ACC_ROWS - N)).reshape(n_seg, pad)
        r = jnp.concatenate([row.reshape(n_seg, seg), padrow], axis=1).reshape(-1)
        c = jnp.concatenate([col.reshape(n_seg, seg), padcol], axis=1).reshape(-1)
        return r, c

    rowp_e, colp_e = _pad_per_tile(2 * N_TILES)
    rowp_f, colp_f = _pad_per_tile(N_TILES)
    zeros128 = jnp.zeros((ACC_ROWS, 128), jnp.float32)

    out = None
    hcat = None
    for i in range(NUM_LAYERS):
        if i == 0:
            x2 = jnp.concatenate([x, x], axis=0)
            agg = _seg_sum_edges(x2, rowp_e, colp_e, zeros128)
            h = x
        else:
            agg = _seg_sum_feat(hcat, rowp_f, colp_f, zeros128)
            h = hcat.reshape(2, N, HID // 2)
        scale = (1.0 + params['eps'][i]).reshape(1, 1)
        lp = params['layers'][i]
        if i < NUM_LAYERS - 1:
            hout = _tc_layer(scale, agg, h, lp, partial_agg=(i == 0))
            hcat = hout.reshape(2 * N, HID // 2)
        else:
            out = _tc_layer(scale, agg, h, lp, partial_agg=False,
                            head=(params['head_W'], params['head_b']))
    return out


# speedup vs baseline: 3.9319x; 1.0143x over previous
import functools

import jax
import jax.numpy as jnp
from jax import lax
from jax.experimental import pallas as pl
from jax.experimental.pallas import tpu as pltpu
from jax.experimental.pallas import tpu_sc as plsc

N = 10000
E = 320000
D_IN = 128
HID = 256
NUM_LAYERS = 3

CHUNK = 128
N_TILES = 16
EP = 327680
N_CHUNKS = EP // CHUNK
CHUNKS_PER_TILE = N_CHUNKS // N_TILES
ACC_ROWS = 12032
ROWS_PER_TILE_INIT = ACC_ROWS // N_TILES
OUT_TILES = 10
ROWS_PER_TILE_OUT = N // OUT_TILES

_MM_PREC = lax.Precision.DEFAULT


def _make_seg_sum(split_edges):
    dh = 128
    mesh = plsc.VectorSubcoreMesh(core_axis_name="c", subcore_axis_name="s")
    chunks_per_tile = CHUNKS_PER_TILE // (2 if split_edges else 1)

    @functools.partial(
        pl.kernel,
        out_type=jax.ShapeDtypeStruct((2, N, dh), jnp.float32),
        mesh=mesh,
        scratch_types=[
            pltpu.VMEM((1, CHUNK), jnp.int32),
            pltpu.VMEM((1, CHUNK), jnp.int32),
            pltpu.VMEM((1, CHUNK), jnp.int32),
            pltpu.VMEM((1, CHUNK), jnp.int32),
            pltpu.VMEM((1, CHUNK), jnp.int32),
            pltpu.VMEM((1, CHUNK), jnp.int32),
            pltpu.VMEM((CHUNK, dh), jnp.float32),
            pltpu.VMEM((CHUNK, dh), jnp.float32),
            pltpu.SemaphoreType.DMA,
            pltpu.SemaphoreType.DMA,
            pltpu.SemaphoreType.DMA,
            pltpu.SemaphoreType.DMA,
            pltpu.VMEM_SHARED((ACC_ROWS, dh), jnp.float32),
        ],
    )
    def seg_sum(h_hbm, row_hbm, col_hbm, zero_hbm, out_hbm,
                rb0, rb1, rr0, rr1, cb0, cb1, gb0, gb1,
                si0, si1, sg0, sg1, acc):
        rb = (rb0, rb1)
        rr = (rr0, rr1)
        cb = (cb0, cb1)
        gb = (gb0, gb1)
        si = (si0, si1)
        sg = (sg0, sg1)
        c = lax.axis_index("c")
        s = lax.axis_index("s")
        z0 = s * ROWS_PER_TILE_INIT
        pltpu.sync_copy(zero_hbm.at[pl.ds(z0, ROWS_PER_TILE_INIT)],
                        acc.at[pl.ds(z0, ROWS_PER_TILE_INIT)])
        plsc.subcore_barrier()

        if split_edges:
            base = (c * N_TILES + s) * (chunks_per_tile * CHUNK)
        else:
            base = s * (chunks_per_tile * CHUNK)
        coff = c * N

        def start_idx(e0, b):
            pltpu.async_copy(row_hbm.at[pl.ds(e0, CHUNK)], rb[b].at[0], si[b])
            pltpu.async_copy(col_hbm.at[pl.ds(e0, CHUNK)], cb[b].at[0], si[b])

        def wait_idx(b):
            pltpu.make_async_copy(
                row_hbm.at[pl.ds(0, CHUNK)], rb[b].at[0], si[b]).wait()
            pltpu.make_async_copy(
                col_hbm.at[pl.ds(0, CHUNK)], cb[b].at[0], si[b]).wait()

        def idxref(b):
            return rr[b]

        def prep(b):
            for q in range(CHUNK // 16):
                rr[b][0, pl.ds(q * 16, 16)] = (
                    rb[b][0, pl.ds(q * 16, 16)] + coff)

        def gather_start(b):
            pltpu.async_copy(h_hbm.at[idxref(b).at[0]], gb[b], sg[b])

        def gather_wait(b):
            pltpu.make_async_copy(h_hbm.at[idxref(b).at[0]], gb[b],
                                  sg[b]).wait()

        def scatter(b):
            pltpu.sync_copy(gb[b], acc.at[cb[b].at[0]], add=True)

        def step(e_cur, b):
            bn = 1 - b
            wait_idx(bn)
            prep(bn)
            gather_wait(b)
            gather_start(bn)
            scatter(b)
            start_idx(e_cur + 2 * CHUNK, b)

        start_idx(base, 0)
        start_idx(base + CHUNK, 1)
        wait_idx(0)
        prep(0)
        gather_start(0)

        @pl.loop(0, (chunks_per_tile - 2) // 2)
        def _(k):
            e0 = base + (2 * k) * CHUNK
            step(e0, 0)
            step(e0 + CHUNK, 1)

        wait_idx(1)
        prep(1)
        gather_wait(0)
        gather_start(1)
        scatter(0)
        gather_wait(1)
        scatter(1)

        plsc.subcore_barrier()

        @pl.when(s < OUT_TILES)
        def _():
            o0 = s * ROWS_PER_TILE_OUT
            pltpu.sync_copy(acc.at[pl.ds(o0, ROWS_PER_TILE_OUT)],
                            out_hbm.at[c, pl.ds(o0, ROWS_PER_TILE_OUT)])

    return seg_sum


@functools.cache
def _seg_sum_kernel(split_edges):
    return _make_seg_sum(split_edges)


def _seg_sum_edges(*args):
    return _seg_sum_kernel(True)(*args)


def _seg_sum_feat(*args):
    return _seg_sum_kernel(False)(*args)


BR = 2000
NB = N // BR
_INV_N = 1.0 / N
_BN_EPS = 1e-5


def _matmul(a, b):
    return jnp.dot(a, b, preferred_element_type=jnp.float32,
                   precision=_MM_PREC)


def _vspec(block, imap):
    return pl.BlockSpec(block, imap, memory_space=pltpu.VMEM)


def _fused_layer_body(partial_agg, with_head):
    def body(*refs):
        if with_head:
            (scale_ref, agg_ref, h_ref, w1_ref, b1_ref, g1_ref, be1_ref,
             w2_ref, b2_ref, go_ref, bo_ref, hw_ref, hb_ref, out_ref,
             t_scr, u_scr, s0, s1, s2, s3) = refs
        else:
            (scale_ref, agg_ref, h_ref, w1_ref, b1_ref, g1_ref, be1_ref,
             w2_ref, b2_ref, go_ref, bo_ref, out_ref,
             t_scr, u_scr, s0, s1, s2, s3) = refs
        p = pl.program_id(0)
        i = pl.program_id(1)
        rows = pl.ds(i * BR, BR)

        @pl.when(p == 0)
        def _():
            s = scale_ref[0, 0]
            if partial_agg:
                z = agg_ref[0] + agg_ref[1] + s * h_ref[...]
            else:
                z = jnp.concatenate(
                    [agg_ref[0] + s * h_ref[0], agg_ref[1] + s * h_ref[1]],
                    axis=1)
            t = _matmul(z, w1_ref[...]) + b1_ref[...]
            t_scr[rows] = t

            @pl.when(i == 0)
            def _():
                s0[...] = jnp.zeros_like(s0)
                s1[...] = jnp.zeros_like(s1)

            s0[...] += jnp.sum(t, axis=0, keepdims=True)
            s1[...] += jnp.sum(t * t, axis=0, keepdims=True)

        @pl.when(p == 1)
        def _():
            mu = s0[...] * _INV_N
            var = s1[...] * _INV_N - mu * mu
            tn = (g1_ref[...] * (t_scr[rows] - mu) * lax.rsqrt(var + _BN_EPS)
                  + be1_ref[...])
            tn = jnp.maximum(tn, 0.0)
            u = _matmul(tn, w2_ref[...]) + b2_ref[...]
            u_scr[rows] = u

            @pl.when(i == 0)
            def _():
                s2[...] = jnp.zeros_like(s2)
                s3[...] = jnp.zeros_like(s3)

            s2[...] += jnp.sum(u, axis=0, keepdims=True)
            s3[...] += jnp.sum(u * u, axis=0, keepdims=True)

        @pl.when(p == 2)
        def _():
            mu = s2[...] * _INV_N
            var = s3[...] * _INV_N - mu * mu
            un = (go_ref[...] * (u_scr[rows] - mu) * lax.rsqrt(var + _BN_EPS)
                  + bo_ref[...])
            un = jnp.maximum(un, 0.0)
            if with_head:
                out_ref[...] = _matmul(un, hw_ref[...]) + hb_ref[...]
            else:
                out_ref[0] = un[:, :HID // 2]
                out_ref[1] = un[:, HID // 2:]
    return body


def _rowb_p0(p, i):
    return (0, jnp.where(p == 0, i, 0), 0)


def _rowb_p0_2d(p, i):
    return (jnp.where(p == 0, i, 0), 0)


def _rowb_p2(p, i):
    return (0, jnp.where(p == 2, i, 0), 0)


def _rowb_p2_2d(p, i):
    return (jnp.where(p == 2, i, 0), 0)


_CONST2 = lambda p, i: (0, 0)
_STAT = pltpu.VMEM((1, HID), jnp.float32)


def _tc_layer(scale, agg, h, lp, partial_agg, head=None):
    d_in = D_IN if partial_agg else HID
    h_spec = (_vspec((BR, D_IN), _rowb_p0_2d) if partial_agg
              else _vspec((2, BR, HID // 2), _rowb_p0))
    args = [scale, agg, h,
            lp['W1'], lp['b1'].reshape(1, HID),
            lp['bn1_g'].reshape(1, HID), lp['bn1_b'].reshape(1, HID),
            lp['W2'], lp['b2'].reshape(1, HID),
            lp['bno_g'].reshape(1, HID), lp['bno_b'].reshape(1, HID)]
    in_specs = [
        pl.BlockSpec(memory_space=pltpu.SMEM),
        _vspec((2, BR, HID // 2), _rowb_p0),
        h_spec,
        _vspec((d_in, HID), _CONST2), _vspec((1, HID), _CONST2),
        _vspec((1, HID), _CONST2), _vspec((1, HID), _CONST2),
        _vspec((HID, HID), _CONST2), _vspec((1, HID), _CONST2),
        _vspec((1, HID), _CONST2), _vspec((1, HID), _CONST2),
    ]
    if head is None:
        out_spec = _vspec((2, BR, HID // 2), _rowb_p2)
        out_shape = jax.ShapeDtypeStruct((2, N, HID // 2), jnp.float32)
    else:
        hw, hb = head
        args += [hw, hb.reshape(1, hw.shape[1])]
        in_specs += [_vspec((HID, hw.shape[1]), _CONST2),
                     _vspec((1, hw.shape[1]), _CONST2)]
        out_spec = _vspec((BR, hw.shape[1]), _rowb_p2_2d)
        out_shape = jax.ShapeDtypeStruct((N, hw.shape[1]), jnp.float32)
    return pl.pallas_call(
        _fused_layer_body(partial_agg, head is not None),
        grid=(3, NB),
        in_specs=in_specs,
        out_specs=out_spec,
        out_shape=out_shape,
        scratch_shapes=[
            pltpu.VMEM((N, HID), jnp.float32),
            pltpu.VMEM((N, HID), jnp.float32),
            _STAT, _STAT, _STAT, _STAT,
        ],
    )(*args)


def kernel(x, edge_index, params):
    row = edge_index[0].astype(jnp.int32)
    col = edge_index[1].astype(jnp.int32)

    def _pad_per_tile(n_seg):
        seg = E // n_seg
        pad = EP // n_seg - seg
        padrow = jnp.zeros((n_seg, pad), jnp.int32)
        padcol = N + (jnp.arange(n_seg * pad, dtype=jnp.int32)
                      % (ACC_ROWS - N)).reshape(n_seg, pad)
        r = jnp.concatenate([row.reshape(n_seg, seg), padrow], axis=1).reshape(-1)
        c = jnp.concatenate([col.reshape(n_seg, seg), padcol], axis=1).reshape(-1)
        return r, c

    rowp_e, colp_e = _pad_per_tile(2 * N_TILES)
    rowp_f, colp_f = _pad_per_tile(N_TILES)
    zeros128 = jnp.zeros((ACC_ROWS, 128), jnp.float32)

    out = None
    hcat = None
    for i in range(NUM_LAYERS):
        if i == 0:
            x2 = jnp.concatenate([x, x], axis=0)
            agg = _seg_sum_edges(x2, rowp_e, colp_e, zeros128)
            h = x
        else:
            agg = _seg_sum_feat(hcat, rowp_f, colp_f, zeros128)
            h = hcat.reshape(2, N, HID // 2)
        scale = (1.0 + params['eps'][i]).reshape(1, 1)
        lp = params['layers'][i]
        if i < NUM_LAYERS - 1:
            hout = _tc_layer(scale, agg, h, lp, partial_agg=(i == 0))
            hcat = hout.reshape(2 * N, HID // 2)
        else:
            out = _tc_layer(scale, agg, h, lp, partial_agg=False,
                            head=(params['head_W'], params['head_b']))
    return out
